# baseline (speedup 1.0000x reference)
import jax
import jax.numpy as jnp
from jax.experimental import pallas as pl

S, L, B, D = 4, 512, 64, 256
_INF = 3.4e38


BB = 2


def _greedy_loss(d):
    rows = jax.lax.broadcasted_iota(jnp.int32, (S, S), 0)
    cols = jax.lax.broadcasted_iota(jnp.int32, (S, S), 1)
    flat_ids = rows * S + cols

    loss_b = jnp.float32(0.0)
    for _ in range(S):
        mval = jnp.min(d)
        idx = jnp.min(jnp.where(d == mval, flat_ids, S * S))
        r = idx // S
        c = idx - r * S
        loss_b = loss_b + mval
        d = jnp.where((rows == r) | (cols == c), _INF, d)
    return loss_b


def _minloss_body(p_ref, g_ref, o_ref):
    b = pl.program_id(0)

    Pblk = p_ref[...]
    Gblk = g_ref[...]

    loss_blk = jnp.float32(0.0)
    for j in range(BB):
        P = Pblk[:, :, j * D:(j + 1) * D]
        G = Gblk[:, j * L:(j + 1) * L, :]

        Ps = [P[s] for s in range(S)]
        Gs = [G[t] for t in range(S)]

        d2 = jnp.stack(
            [jnp.stack([jnp.sum((Ps[s] - Gs[t]) ** 2) for t in range(S)])
             for s in range(S)]
        )
        d = jnp.sqrt(jnp.maximum(d2, 0.0))
        loss_blk = loss_blk + _greedy_loss(d)

    @pl.when(b == 0)
    def _init():
        o_ref[...] = jnp.zeros_like(o_ref)

    o_ref[...] = o_ref[...] + loss_blk


def kernel(predictions, ground_truths):
    pred_r = predictions.reshape(S, L, B * D)
    gt_r = ground_truths.reshape(S, B * L, D)

    out = pl.pallas_call(
        _minloss_body,
        grid=(B // BB,),
        in_specs=[
            pl.BlockSpec((S, L, BB * D), lambda b: (0, 0, b)),
            pl.BlockSpec((S, BB * L, D), lambda b: (0, b, 0)),
        ],
        out_specs=pl.BlockSpec((1, 1), lambda b: (0, 0)),
        out_shape=jax.ShapeDtypeStruct((1, 1), jnp.float32),
    )(pred_r, gt_r)
    return out[0, 0]
